# Initial kernel scaffold; baseline (speedup 1.0000x reference)
import jax
import jax.numpy as jnp
from jax.experimental import pallas as pl


def kernel(x, target, pick_frequency):
    raise NotImplementedError("write your pallas kernel here")


# speedup vs baseline: 1.4055x; 1.4055x over previous
import math

import jax
import jax.numpy as jnp
from jax import lax
from jax.experimental import pallas as pl
from jax.experimental.pallas import tpu as pltpu

_K = 8


def _body(x_ref, t_ref, pf_ref, sel_ref, ml_ref):
    xb = x_ref[0]
    tb = t_ref[0]
    k, d, hw = xb.shape
    inv_n = 1.0 / (d * hw)
    penalty = math.log(_K, 2) / hw

    diff = xb - tb[None, :, :]
    part = jnp.sum(diff * diff, axis=2)
    ssq = jnp.sum(part, axis=1, keepdims=True)
    loss = ssq * inv_n + penalty

    pf = pf_ref[:, 0:1]
    scaled = loss * pf
    m = jnp.min(scaled)
    iota = lax.broadcasted_iota(jnp.int32, (k, 1), 0)
    idx = jnp.min(jnp.where(scaled == m, iota, k))
    min_loss = jnp.sum(jnp.where(iota == idx, loss, 0.0))

    ml_ref[0] = jnp.full((1, 128), min_loss, jnp.float32)
    sel_ref[0] = x_ref[0, idx]


def kernel(x, target, pick_frequency):
    B, C, H, W = x.shape
    D = C // _K
    HW = H * W
    x4 = x.reshape(B, _K, D, HW)
    t3 = target.reshape(B, D, HW)
    pf2 = jnp.broadcast_to(pick_frequency[:, None], (_K, 128))

    sel, ml = pl.pallas_call(
        _body,
        grid=(B,),
        in_specs=[
            pl.BlockSpec((1, _K, D, HW), lambda b: (b, 0, 0, 0)),
            pl.BlockSpec((1, D, HW), lambda b: (b, 0, 0)),
            pl.BlockSpec((_K, 128), lambda b: (0, 0)),
        ],
        out_specs=[
            pl.BlockSpec((1, D, HW), lambda b: (b, 0, 0)),
            pl.BlockSpec((1, 1, 128), lambda b: (b, 0, 0)),
        ],
        out_shape=[
            jax.ShapeDtypeStruct((B, D, HW), jnp.float32),
            jax.ShapeDtypeStruct((B, 1, 128), jnp.float32),
        ],
        compiler_params=pltpu.CompilerParams(
            dimension_semantics=("arbitrary",),
        ),
    )(x4, t3, pf2)

    selected = sel.reshape(B, D, H, W)
    min_loss = ml[:, 0, 0]
    return selected, min_loss


# speedup vs baseline: 7.3816x; 5.2521x over previous
import math

import jax
import jax.numpy as jnp
from jax.experimental import pallas as pl
from jax.experimental.pallas import tpu as pltpu

_K = 8


def _body(pf_ref, x_ref, t_ref, sel_ref, ml_ref):
    _, h, w, d = t_ref.shape
    inv_n = 1.0 / (h * w * d)
    penalty = math.log(_K, 2) / (h * w)

    tb = t_ref[0]

    best_scaled = jnp.float32(jnp.inf)
    best_loss = jnp.float32(0.0)
    best_idx = jnp.int32(0)
    for k in range(_K):
        chunk = x_ref[0, :, :, k * d:(k + 1) * d]
        diff = chunk - tb
        loss_k = jnp.sum(diff * diff) * inv_n + penalty
        scaled_k = loss_k * pf_ref[0, k]
        better = scaled_k < best_scaled
        best_scaled = jnp.where(better, scaled_k, best_scaled)
        best_loss = jnp.where(better, loss_k, best_loss)
        best_idx = jnp.where(better, jnp.int32(k), best_idx)

    ml_ref[0] = jnp.full((1, 128), best_loss, jnp.float32)
    for k in range(_K):
        @pl.when(best_idx == k)
        def _():
            sel_ref[0] = x_ref[0, :, :, k * d:(k + 1) * d]


def kernel(x, target, pick_frequency):
    B, C, H, W = x.shape
    D = C // _K
    xt = jnp.transpose(x, (0, 2, 3, 1))
    tt = jnp.transpose(target, (0, 2, 3, 1))
    pf = pick_frequency.reshape(1, _K)

    sel, ml = pl.pallas_call(
        _body,
        grid=(B,),
        in_specs=[
            pl.BlockSpec(memory_space=pltpu.SMEM),
            pl.BlockSpec((1, H, W, C), lambda b: (b, 0, 0, 0)),
            pl.BlockSpec((1, H, W, D), lambda b: (b, 0, 0, 0)),
        ],
        out_specs=[
            pl.BlockSpec((1, H, W, D), lambda b: (b, 0, 0, 0)),
            pl.BlockSpec((1, 1, 128), lambda b: (b, 0, 0)),
        ],
        out_shape=[
            jax.ShapeDtypeStruct((B, H, W, D), jnp.float32),
            jax.ShapeDtypeStruct((B, 1, 128), jnp.float32),
        ],
        compiler_params=pltpu.CompilerParams(
            dimension_semantics=("arbitrary",),
        ),
    )(pf, xt, tt)

    selected = jnp.transpose(sel, (0, 3, 1, 2))
    min_loss = ml[:, 0, 0]
    return selected, min_loss


# speedup vs baseline: 7.3822x; 1.0001x over previous
import math

import jax
import jax.numpy as jnp
from jax.experimental import pallas as pl
from jax.experimental.pallas import tpu as pltpu

_K = 8


def _body(pf_ref, x_ref, t_ref, sel_ref, ml_ref):
    _, h, w, d = t_ref.shape
    inv_n = 1.0 / (h * w * d)
    penalty = math.log(_K, 2) / (h * w)

    tb = t_ref[0]

    best_scaled = jnp.float32(jnp.inf)
    best_loss = jnp.float32(0.0)
    best_idx = jnp.int32(0)
    for k in range(_K):
        chunk = x_ref[0, :, :, k * d:(k + 1) * d]
        diff = chunk - tb
        loss_k = jnp.sum(diff * diff) * inv_n + penalty
        scaled_k = loss_k * pf_ref[0, k]
        better = scaled_k < best_scaled
        best_scaled = jnp.where(better, scaled_k, best_scaled)
        best_loss = jnp.where(better, loss_k, best_loss)
        best_idx = jnp.where(better, jnp.int32(k), best_idx)

    ml_ref[0] = jnp.full((1, 128), best_loss, jnp.float32)
    for k in range(_K):
        @pl.when(best_idx == k)
        def _():
            sel_ref[0] = x_ref[0, :, :, k * d:(k + 1) * d]


def kernel(x, target, pick_frequency):
    B, C, H, W = x.shape
    D = C // _K
    xt = jnp.transpose(x, (0, 2, 3, 1))
    tt = jnp.transpose(target, (0, 2, 3, 1))
    pf = pick_frequency.reshape(1, _K)

    sel, ml = pl.pallas_call(
        _body,
        grid=(B,),
        in_specs=[
            pl.BlockSpec(memory_space=pltpu.SMEM),
            pl.BlockSpec((1, H, W, C), lambda b: (b, 0, 0, 0)),
            pl.BlockSpec((1, H, W, D), lambda b: (b, 0, 0, 0)),
        ],
        out_specs=[
            pl.BlockSpec((1, H, W, D), lambda b: (b, 0, 0, 0)),
            pl.BlockSpec((1, 1, 128), lambda b: (b, 0, 0)),
        ],
        out_shape=[
            jax.ShapeDtypeStruct((B, H, W, D), jnp.float32),
            jax.ShapeDtypeStruct((B, 1, 128), jnp.float32),
        ],
        compiler_params=pltpu.CompilerParams(
            dimension_semantics=("parallel",),
        ),
    )(pf, xt, tt)

    selected = jnp.transpose(sel, (0, 3, 1, 2))
    min_loss = ml[:, 0, 0]
    return selected, min_loss


# speedup vs baseline: 8.1131x; 1.0990x over previous
import math

import jax
import jax.numpy as jnp
from jax.experimental import pallas as pl
from jax.experimental.pallas import tpu as pltpu

_K = 8
_BS = 2


def _body(pf_ref, x_ref, t_ref, sel_ref, ml_ref):
    _, h, w, d = t_ref.shape
    inv_n = 1.0 / (h * w * d)
    penalty = math.log(_K, 2) / (h * w)

    for s in range(_BS):
        tb = t_ref[s]

        best_scaled = jnp.float32(jnp.inf)
        best_loss = jnp.float32(0.0)
        best_idx = jnp.int32(0)
        for k in range(_K):
            chunk = x_ref[s, :, :, k * d:(k + 1) * d]
            diff = chunk - tb
            loss_k = jnp.sum(diff * diff) * inv_n + penalty
            scaled_k = loss_k * pf_ref[0, k]
            better = scaled_k < best_scaled
            best_scaled = jnp.where(better, scaled_k, best_scaled)
            best_loss = jnp.where(better, loss_k, best_loss)
            best_idx = jnp.where(better, jnp.int32(k), best_idx)

        ml_ref[s] = jnp.full((1, 128), best_loss, jnp.float32)
        for k in range(_K):
            @pl.when(best_idx == k)
            def _():
                sel_ref[s] = x_ref[s, :, :, k * d:(k + 1) * d]


def kernel(x, target, pick_frequency):
    B, C, H, W = x.shape
    D = C // _K
    xt = jnp.transpose(x, (0, 2, 3, 1))
    tt = jnp.transpose(target, (0, 2, 3, 1))
    pf = pick_frequency.reshape(1, _K)

    sel, ml = pl.pallas_call(
        _body,
        grid=(B // _BS,),
        in_specs=[
            pl.BlockSpec(memory_space=pltpu.SMEM),
            pl.BlockSpec((_BS, H, W, C), lambda b: (b, 0, 0, 0)),
            pl.BlockSpec((_BS, H, W, D), lambda b: (b, 0, 0, 0)),
        ],
        out_specs=[
            pl.BlockSpec((_BS, H, W, D), lambda b: (b, 0, 0, 0)),
            pl.BlockSpec((_BS, 1, 128), lambda b: (b, 0, 0)),
        ],
        out_shape=[
            jax.ShapeDtypeStruct((B, H, W, D), jnp.float32),
            jax.ShapeDtypeStruct((B, 1, 128), jnp.float32),
        ],
        compiler_params=pltpu.CompilerParams(
            dimension_semantics=("parallel",),
        ),
    )(pf, xt, tt)

    selected = jnp.transpose(sel, (0, 3, 1, 2))
    min_loss = ml[:, 0, 0]
    return selected, min_loss
